# Initial kernel scaffold; baseline (speedup 1.0000x reference)
import jax
import jax.numpy as jnp
from jax.experimental import pallas as pl


def kernel(input, rois):
    raise NotImplementedError("write your pallas kernel here")


# speedup vs baseline: 5.4488x; 5.4488x over previous
import functools

import jax
import jax.numpy as jnp
from jax import lax
from jax.experimental import pallas as pl
from jax.experimental.pallas import tpu as pltpu
from jax.experimental.pallas import tpu_sc as plsc

N, C, H, W = 2, 256, 100, 152
R = 512
PH = PW = 7
SR = 2
SCALE = 0.125

NC, NS, L = 2, 16, 16
NW = NC * NS
OUT_ROWS = R * PH * PW
G = 16
GROUPS = OUT_ROWS // G
GROUPS_PER_W = GROUPS // NW
SLOTS = SR * SR * 4


def _mesh():
    return plsc.VectorSubcoreMesh(
        core_axis_name="c", subcore_axis_name="s", num_cores=NC, num_subcores=NS
    )


@functools.partial(
    pl.kernel,
    out_type=jax.ShapeDtypeStruct((OUT_ROWS * C,), jnp.float32),
    mesh=_mesh(),
    compiler_params=pltpu.CompilerParams(needs_layout_passes=False),
    scratch_types=[
        pltpu.VMEM((R * 5,), jnp.float32),
        pltpu.VMEM((8 * L,), jnp.int32),
        pltpu.VMEM((8 * L,), jnp.int32),
        pltpu.VMEM((SLOTS * L,), jnp.float32),
        pltpu.VMEM((SLOTS * L, C), jnp.float32),
        pltpu.VMEM((G * C,), jnp.float32),
        pltpu.SemaphoreType.DMA,
    ],
)
def _roi_pool_sc(feat_hbm, rois_hbm, out_hbm, rois_v, idx0_v, idx1_v, w_v,
                 buf_v, ostage_v, sem):
    wid = lax.axis_index("s") * NC + lax.axis_index("c")
    pltpu.sync_copy(rois_hbm, rois_v)

    def group_body(g, _):
        base = wid * (GROUPS_PER_W * G) + g * G
        orv = base + lax.iota(jnp.int32, L)
        r = lax.div(orv, PH * PW)
        rem = lax.rem(orv, PH * PW)
        ph = lax.div(rem, PW)
        pw = lax.rem(rem, PW)

        r5 = r * 5
        col = lambda c: plsc.load_gather(rois_v, [r5 + c])
        b_i = col(0).astype(jnp.int32)
        x1 = col(1) * SCALE - 0.5
        y1 = col(2) * SCALE - 0.5
        x2 = col(3) * SCALE - 0.5
        y2 = col(4) * SCALE - 0.5
        bw = jnp.maximum(x2 - x1, 1.0) * (1.0 / PW)
        bh = jnp.maximum(y2 - y1, 1.0) * (1.0 / PH)
        base_row = b_i * (H * W)
        ph_f = ph.astype(jnp.float32)
        pw_f = pw.astype(jnp.float32)

        wy, ry = [], []
        for s in range(SR):
            ys = y1 + (ph_f + (0.5 + s) / SR) * bh
            vy = jnp.where((ys > -1.0) & (ys < float(H)), 0.5, 0.0)
            yc = jnp.clip(ys, 0.0, float(H - 1))
            y0i = yc.astype(jnp.int32)
            ly = yc - y0i.astype(jnp.float32)
            wy.append([(1.0 - ly) * vy, ly * vy])
            ry.append([y0i * W, jnp.minimum(y0i + 1, H - 1) * W])
        wx, rx = [], []
        for t in range(SR):
            xs = x1 + (pw_f + (0.5 + t) / SR) * bw
            vx = jnp.where((xs > -1.0) & (xs < float(W)), 0.5, 0.0)
            xc = jnp.clip(xs, 0.0, float(W - 1))
            x0i = xc.astype(jnp.int32)
            lx = xc - x0i.astype(jnp.float32)
            wx.append([(1.0 - lx) * vx, lx * vx])
            rx.append([x0i, jnp.minimum(x0i + 1, W - 1)])

        k = 0
        for s in range(SR):
            for t in range(SR):
                for i in range(2):
                    for j in range(2):
                        idx = base_row + ry[s][i] + rx[t][j]
                        if k < 8:
                            idx0_v[pl.ds(k * L, L)] = idx
                        else:
                            idx1_v[pl.ds((k - 8) * L, L)] = idx
                        w_v[pl.ds(k * L, L)] = wy[s][i] * wx[t][j]
                        k += 1

        cp0 = pltpu.async_copy(feat_hbm.at[idx0_v], buf_v.at[pl.ds(0, 8 * L)],
                               sem)
        cp1 = pltpu.async_copy(feat_hbm.at[idx1_v],
                               buf_v.at[pl.ds(8 * L, 8 * L)], sem)
        cp0.wait()
        cp1.wait()

        def o_body(o, _):
            def k_body(kk, accs):
                m = kk * L + o
                wv = plsc.load_gather(w_v, [lax.broadcast(m, (L,))])
                return tuple(
                    accs[j] + wv * buf_v[m, pl.ds(j * L, L)]
                    for j in range(C // L)
                )

            accs = lax.fori_loop(
                0, SLOTS, k_body,
                tuple(jnp.zeros((L,), jnp.float32) for _ in range(C // L)),
            )
            for j in range(C // L):
                ostage_v[pl.ds(o * C + j * L, L)] = accs[j]
            return 0

        lax.fori_loop(0, G, o_body, 0)
        pltpu.sync_copy(ostage_v, out_hbm.at[pl.ds(base * C, G * C)])
        return 0

    lax.fori_loop(0, GROUPS_PER_W, group_body, 0)


def kernel(input, rois):
    feat = jnp.transpose(input, (0, 2, 3, 1)).reshape(N * H * W, C)
    out_flat = _roi_pool_sc(feat, rois.reshape(-1))
    return out_flat.reshape(R, PH, PW, C).transpose(0, 3, 1, 2)
